# baseline (speedup 1.0000x reference)
import jax
import jax.numpy as jnp
from jax.experimental import pallas as pl

NUM_SEL = 300


def _prob_body(logits_ref, pmT_ref, prob_ref):
    sig = jax.nn.sigmoid(logits_ref[0])
    prob_ref[0] = jnp.dot(sig, pmT_ref[...],
                          preferred_element_type=jnp.float32)


def kernel(pred_logits, pred_boxes, target_sizes, pos_map):
    B, NQ, T = pred_logits.shape
    C = pos_map.shape[0]
    Cp = 64
    pmT = jnp.zeros((T, Cp), pos_map.dtype).at[:, :C].set(pos_map.T)

    prob = pl.pallas_call(
        _prob_body,
        grid=(B,),
        in_specs=[
            pl.BlockSpec((1, NQ, T), lambda b: (b, 0, 0)),
            pl.BlockSpec((T, Cp), lambda b: (0, 0)),
        ],
        out_specs=pl.BlockSpec((1, NQ, Cp), lambda b: (b, 0, 0)),
        out_shape=jax.ShapeDtypeStruct((B, NQ, Cp), jnp.float32),
    )(pred_logits, pmT)

    flat = prob[..., :C].reshape(B, NQ * C)
    topk_values, topk_indexes = jax.lax.top_k(flat, NUM_SEL)
    scores = topk_values
    topk_boxes = topk_indexes // C
    labels = topk_indexes % C

    cx, cy, w, h = jnp.split(pred_boxes, 4, axis=-1)
    boxes = jnp.concatenate(
        [cx - 0.5 * w, cy - 0.5 * h, cx + 0.5 * w, cy + 0.5 * h], axis=-1)
    boxes = jnp.take_along_axis(boxes, topk_boxes[..., None], axis=1)
    img_h = target_sizes[:, 0]
    img_w = target_sizes[:, 1]
    scale_fct = jnp.stack([img_w, img_h, img_w, img_h], axis=1)
    boxes = boxes * scale_fct[:, None, :]
    return scores, labels, boxes


# speedup vs baseline: 1.7956x; 1.7956x over previous
import jax
import jax.numpy as jnp
from jax.experimental import pallas as pl

NUM_SEL = 300
P = 304
CP = 64


def _dotT(a, b):
    return jax.lax.dot_general(a, b, (((0,), (0,)), ((), ())),
                               preferred_element_type=jnp.float32,
                               precision=jax.lax.Precision.HIGHEST)


def _body(logits_ref, pmT_ref, boxes_ref, scale_ref,
          scores_ref, labels_ref, boxes_out_ref):
    nq_raw = logits_ref.shape[1]

    sig = jax.nn.sigmoid(logits_ref[0])
    prob = jnp.dot(sig, pmT_ref[...],
                   preferred_element_type=jnp.float32)

    nq = (nq_raw + 63) // 64 * 64
    npad = nq - nq_raw
    prob = jnp.concatenate(
        [prob, jnp.full((npad, CP), -1.0, jnp.float32)], axis=0)

    bits = jax.lax.bitcast_convert_type(prob, jnp.int32)

    def bs_body(_, carry):
        lo, hi = carry
        mid = (lo + hi) // 2
        cnt = jnp.sum((bits >= mid).astype(jnp.int32))
        take = cnt >= NUM_SEL
        return (jnp.where(take, mid, lo), jnp.where(take, hi, mid))

    lo, _ = jax.lax.fori_loop(0, 31, bs_body,
                              (jnp.int32(0), jnp.int32(0x3F800001)))
    thr = lo

    qi = jax.lax.broadcasted_iota(jnp.int32, (nq, CP), 0)
    ci = jax.lax.broadcasted_iota(jnp.int32, (nq, CP), 1)
    flat = qi * CP + ci
    cnt_gt = jnp.sum((bits > thr).astype(jnp.int32))
    tied = bits == thr

    def ix_body(_, carry):
        lo2, hi2 = carry
        mid = (lo2 + hi2) // 2
        cnt = cnt_gt + jnp.sum((tied & (flat < mid)).astype(jnp.int32))
        take = cnt >= NUM_SEL
        return (jnp.where(take, lo2, mid), jnp.where(take, mid, hi2))

    _, kcut = jax.lax.fori_loop(0, 17, ix_body,
                                (jnp.int32(0), jnp.int32(nq * CP)))

    sel = (bits > thr) | (tied & (flat < kcut))
    self32 = sel.astype(jnp.float32)

    rowcnt = jnp.sum(self32, axis=1, keepdims=True)
    tri = (jax.lax.broadcasted_iota(jnp.int32, (nq, nq), 1)
           <= jax.lax.broadcasted_iota(jnp.int32, (nq, nq), 0)
           ).astype(jnp.float32)
    rpi = jnp.dot(tri, rowcnt,
                  preferred_element_type=jnp.float32)
    rp = rpi - rowcnt
    up = (jax.lax.broadcasted_iota(jnp.int32, (CP, CP), 0)
          < jax.lax.broadcasted_iota(jnp.int32, (CP, CP), 1)
          ).astype(jnp.float32)
    ir = jnp.dot(self32, up,
                 preferred_element_type=jnp.float32)

    piota_l = jax.lax.broadcasted_iota(jnp.int32, (nq, P), 1).astype(jnp.float32)
    m = (rpi <= piota_l).astype(jnp.float32)
    q_p = jnp.sum(m, axis=0).reshape(1, P)

    row_oh = (jax.lax.broadcasted_iota(jnp.int32, (nq, P), 0)
              .astype(jnp.float32) == q_p).astype(jnp.float32)

    bx = jnp.concatenate(
        [boxes_ref[0], jnp.zeros((npad, 4), jnp.float32)], axis=0)
    data = jnp.concatenate([prob, ir, self32, rp, bx], axis=1)
    y = _dotT(row_oh, data)
    yval = y[:, :CP]
    yir = y[:, CP:2 * CP]
    ysel = y[:, 2 * CP:3 * CP]
    yrp = y[:, 3 * CP:3 * CP + 1]
    ybox = y[:, 3 * CP + 1:3 * CP + 5]

    piota_c = jax.lax.broadcasted_iota(jnp.int32, (P, 1), 0).astype(jnp.float32)
    j_p = piota_c - yrp
    col_oh = ((yir == j_p) & (ysel > 0.5)).astype(jnp.float32)

    ciota = jax.lax.broadcasted_iota(jnp.int32, (P, CP), 1).astype(jnp.float32)
    val_c = jnp.sum(yval * col_oh, axis=1, keepdims=True)
    lab_c = jnp.sum(ciota * col_oh, axis=1, keepdims=True)

    pad = piota_c >= float(NUM_SEL)
    val_c = jnp.where(pad, -1.0 - piota_c, val_c)

    val_r = jnp.sum(val_c * (jax.lax.broadcasted_iota(jnp.int32, (P, P), 0)
                             == jax.lax.broadcasted_iota(jnp.int32, (P, P), 1)
                             ).astype(jnp.float32), axis=0).reshape(1, P)
    gt = (val_r > val_c).astype(jnp.float32)
    jlt = (jax.lax.broadcasted_iota(jnp.int32, (P, P), 1)
           < jax.lax.broadcasted_iota(jnp.int32, (P, P), 0))
    tie = ((val_r == val_c) & jlt).astype(jnp.float32)
    rank = jnp.sum(gt + tie, axis=1, keepdims=True)

    fin_oh = (rank == jax.lax.broadcasted_iota(jnp.int32, (P, P), 1)
              .astype(jnp.float32)).astype(jnp.float32)

    scores = jnp.sum(fin_oh * val_c, axis=0).reshape(1, P)
    labels = jnp.sum(fin_oh * lab_c, axis=0).reshape(1, P)

    cx, cy, w, h = (ybox[:, 0:1], ybox[:, 1:2], ybox[:, 2:3], ybox[:, 3:4])
    xyxy = jnp.concatenate(
        [cx - 0.5 * w, cy - 0.5 * h, cx + 0.5 * w, cy + 0.5 * h], axis=1)
    xyxy = xyxy * scale_ref[0]
    boxes_fin = _dotT(fin_oh, xyxy)

    scores_ref[0] = scores
    labels_ref[0] = labels.astype(jnp.int32)
    boxes_out_ref[0] = boxes_fin


def kernel(pred_logits, pred_boxes, target_sizes, pos_map):
    B, NQ, T = pred_logits.shape
    C = pos_map.shape[0]
    pmT = jnp.zeros((T, CP), pos_map.dtype).at[:, :C].set(pos_map.T)
    img_h = target_sizes[:, 0]
    img_w = target_sizes[:, 1]
    scale = jnp.stack([img_w, img_h, img_w, img_h], axis=1).reshape(B, 1, 4)

    scores, labels, boxes = pl.pallas_call(
        _body,
        grid=(B,),
        in_specs=[
            pl.BlockSpec((1, NQ, T), lambda b: (b, 0, 0)),
            pl.BlockSpec((T, CP), lambda b: (0, 0)),
            pl.BlockSpec((1, NQ, 4), lambda b: (b, 0, 0)),
            pl.BlockSpec((1, 1, 4), lambda b: (b, 0, 0)),
        ],
        out_specs=[
            pl.BlockSpec((1, 1, P), lambda b: (b, 0, 0)),
            pl.BlockSpec((1, 1, P), lambda b: (b, 0, 0)),
            pl.BlockSpec((1, P, 4), lambda b: (b, 0, 0)),
        ],
        out_shape=[
            jax.ShapeDtypeStruct((B, 1, P), jnp.float32),
            jax.ShapeDtypeStruct((B, 1, P), jnp.int32),
            jax.ShapeDtypeStruct((B, P, 4), jnp.float32),
        ],
    )(pred_logits, pmT, pred_boxes, scale)

    return (scores[:, 0, :NUM_SEL], labels[:, 0, :NUM_SEL],
            boxes[:, :NUM_SEL, :])
